# initial kernel scaffold (unmeasured)
import jax
import jax.numpy as jnp
from jax import lax
from jax.experimental import pallas as pl
from jax.experimental.pallas import tpu as pltpu


def kernel(
    x,
):
    def body(*refs):
        pass

    out_shape = jax.ShapeDtypeStruct(..., jnp.float32)
    return pl.pallas_call(body, out_shape=out_shape)(...)


# baseline (device time: 18560 ns/iter reference)
import jax
import jax.numpy as jnp
from jax import lax
from jax.experimental import pallas as pl
from jax.experimental.pallas import tpu as pltpu

M = 1024
HALF = 512


def kernel(x):
    def body(x_ref, out_ref, send_buf, recv_buf, send_sem, recv_sem):
        my_x = lax.axis_index("x")
        my_y = lax.axis_index("y")
        my_z = lax.axis_index("z")
        peer = (my_x, 1 - my_y, my_z)

        barrier = pltpu.get_barrier_semaphore()
        pl.semaphore_signal(
            barrier, inc=1, device_id=peer, device_id_type=pl.DeviceIdType.MESH
        )
        pl.semaphore_wait(barrier, 1)

        @pl.when(my_y == 0)
        def _():
            send_buf[:, :] = x_ref[0, :, HALF:].astype(jnp.bfloat16)

        @pl.when(my_y == 1)
        def _():
            send_buf[:, :] = x_ref[0, :, :HALF].astype(jnp.bfloat16)

        rdma = pltpu.make_async_remote_copy(
            src_ref=send_buf,
            dst_ref=recv_buf,
            send_sem=send_sem,
            recv_sem=recv_sem,
            device_id=peer,
            device_id_type=pl.DeviceIdType.MESH,
        )
        rdma.start()
        rdma.wait()

        @pl.when(my_y == 0)
        def _():
            out_ref[:, :] = x_ref[0, :, :HALF].astype(jnp.bfloat16) + recv_buf[:, :]

        @pl.when(my_y == 1)
        def _():
            out_ref[:, :] = x_ref[0, :, HALF:].astype(jnp.bfloat16) + recv_buf[:, :]

    return pl.pallas_call(
        body,
        out_shape=jax.ShapeDtypeStruct((M, HALF), jnp.bfloat16),
        in_specs=[pl.BlockSpec(memory_space=pltpu.VMEM)],
        out_specs=pl.BlockSpec(memory_space=pltpu.VMEM),
        scratch_shapes=[
            pltpu.VMEM((M, HALF), jnp.bfloat16),
            pltpu.VMEM((M, HALF), jnp.bfloat16),
            pltpu.SemaphoreType.DMA,
            pltpu.SemaphoreType.DMA,
        ],
        compiler_params=pltpu.CompilerParams(collective_id=0),
    )(x)


# device time: 18433 ns/iter; 1.0069x vs baseline; 1.0069x over previous
import jax
import jax.numpy as jnp
from jax import lax
from jax.experimental import pallas as pl
from jax.experimental.pallas import tpu as pltpu

M = 1024
HALF = 512
NCHUNK = 2
ROWS = M // NCHUNK


def kernel(x):
    def body(x_ref, out_ref, send_buf, recv_buf, send_sems, recv_sems):
        my_x = lax.axis_index("x")
        my_y = lax.axis_index("y")
        my_z = lax.axis_index("z")
        peer = (my_x, 1 - my_y, my_z)

        barrier = pltpu.get_barrier_semaphore()
        pl.semaphore_signal(
            barrier, inc=1, device_id=peer, device_id_type=pl.DeviceIdType.MESH
        )
        pl.semaphore_wait(barrier, 1)

        peer_col = (1 - my_y) * HALF
        my_col = my_y * HALF

        rdmas = []
        for r in range(NCHUNK):
            rows = pl.ds(r * ROWS, ROWS)
            send_buf[r] = x_ref[0, rows, pl.ds(peer_col, HALF)].astype(jnp.bfloat16)
            rdma = pltpu.make_async_remote_copy(
                src_ref=send_buf.at[r],
                dst_ref=recv_buf.at[r],
                send_sem=send_sems.at[r],
                recv_sem=recv_sems.at[r],
                device_id=peer,
                device_id_type=pl.DeviceIdType.MESH,
            )
            rdma.start()
            rdmas.append(rdma)

        out_ref[:, :] = x_ref[0, :, pl.ds(my_col, HALF)].astype(jnp.bfloat16)

        for r in range(NCHUNK):
            rows = pl.ds(r * ROWS, ROWS)
            rdmas[r].wait()
            out_ref[rows, :] = out_ref[rows, :] + recv_buf[r]

    return pl.pallas_call(
        body,
        out_shape=jax.ShapeDtypeStruct((M, HALF), jnp.bfloat16),
        in_specs=[pl.BlockSpec(memory_space=pltpu.VMEM)],
        out_specs=pl.BlockSpec(memory_space=pltpu.VMEM),
        scratch_shapes=[
            pltpu.VMEM((NCHUNK, ROWS, HALF), jnp.bfloat16),
            pltpu.VMEM((NCHUNK, ROWS, HALF), jnp.bfloat16),
            pltpu.SemaphoreType.DMA((NCHUNK,)),
            pltpu.SemaphoreType.DMA((NCHUNK,)),
        ],
        compiler_params=pltpu.CompilerParams(collective_id=0),
    )(x)


# device time: 16920 ns/iter; 1.0969x vs baseline; 1.0894x over previous
import jax
import jax.numpy as jnp
from jax import lax
from jax.experimental import pallas as pl
from jax.experimental.pallas import tpu as pltpu

M = 1024
HALF = 512
ROWS = 512
NCH = 4
CROWS = ROWS // NCH


def kernel(x):
    def body(
        x_ref,
        out_ref,
        in_peer,
        in_mine,
        send_y,
        recv_y,
        dsem_p,
        dsem_m,
        ysend,
        yrecv,
        zsend,
        zrecv,
    ):
        my_x = lax.axis_index("x")
        my_y = lax.axis_index("y")
        my_z = lax.axis_index("z")
        ypeer = (my_x, 1 - my_y, my_z)
        znb = (my_x, my_y, 1 - my_z)

        row0 = my_z * ROWS
        other0 = (1 - my_z) * ROWS
        my_col = my_y * HALF
        peer_col = (1 - my_y) * HALF

        dmas_p = []
        dmas_m = []
        for c in range(NCH):
            rows = pl.ds(row0 + c * CROWS, CROWS)
            dp = pltpu.make_async_copy(
                x_ref.at[0, rows, pl.ds(peer_col, HALF)], in_peer.at[c], dsem_p.at[c]
            )
            dp.start()
            dmas_p.append(dp)
            dm = pltpu.make_async_copy(
                x_ref.at[0, rows, pl.ds(my_col, HALF)], in_mine.at[c], dsem_m.at[c]
            )
            dm.start()
            dmas_m.append(dm)

        barrier = pltpu.get_barrier_semaphore()
        for nbr in (ypeer, znb):
            pl.semaphore_signal(
                barrier, inc=1, device_id=nbr, device_id_type=pl.DeviceIdType.MESH
            )
        pl.semaphore_wait(barrier, 2)

        rdmas_y = []
        for c in range(NCH):
            dmas_p[c].wait()
            send_y[c] = in_peer[c].astype(jnp.bfloat16)
            ry = pltpu.make_async_remote_copy(
                src_ref=send_y.at[c],
                dst_ref=recv_y.at[c],
                send_sem=ysend.at[c],
                recv_sem=yrecv.at[c],
                device_id=ypeer,
                device_id_type=pl.DeviceIdType.MESH,
            )
            ry.start()
            rdmas_y.append(ry)

        rdmas_z = []
        for c in range(NCH):
            dmas_m[c].wait()
            rdmas_y[c].wait()
            rows = pl.ds(row0 + c * CROWS, CROWS)
            out_ref[rows, :] = in_mine[c].astype(jnp.bfloat16) + recv_y[c]
            rz = pltpu.make_async_remote_copy(
                src_ref=out_ref.at[rows],
                dst_ref=out_ref.at[rows],
                send_sem=zsend.at[c],
                recv_sem=zrecv.at[c],
                device_id=znb,
                device_id_type=pl.DeviceIdType.MESH,
            )
            rz.start()
            rdmas_z.append(rz)

        for c in range(NCH):
            rdmas_z[c].wait()

    return pl.pallas_call(
        body,
        out_shape=jax.ShapeDtypeStruct((M, HALF), jnp.bfloat16),
        in_specs=[pl.BlockSpec(memory_space=pl.ANY)],
        out_specs=pl.BlockSpec(memory_space=pltpu.VMEM),
        scratch_shapes=[
            pltpu.VMEM((NCH, CROWS, HALF), jnp.float32),
            pltpu.VMEM((NCH, CROWS, HALF), jnp.float32),
            pltpu.VMEM((NCH, CROWS, HALF), jnp.bfloat16),
            pltpu.VMEM((NCH, CROWS, HALF), jnp.bfloat16),
            pltpu.SemaphoreType.DMA((NCH,)),
            pltpu.SemaphoreType.DMA((NCH,)),
            pltpu.SemaphoreType.DMA((NCH,)),
            pltpu.SemaphoreType.DMA((NCH,)),
            pltpu.SemaphoreType.DMA((NCH,)),
            pltpu.SemaphoreType.DMA((NCH,)),
        ],
        compiler_params=pltpu.CompilerParams(collective_id=0),
    )(x)


# device time: 16720 ns/iter; 1.1100x vs baseline; 1.0120x over previous
import jax
import jax.numpy as jnp
from jax import lax
from jax.experimental import pallas as pl
from jax.experimental.pallas import tpu as pltpu

M = 1024
HALF = 512
ROWS = 512
NCH = 8
CROWS = ROWS // NCH


def kernel(x):
    def body(
        x_ref,
        out_ref,
        in_peer,
        in_mine,
        send_y,
        recv_y,
        sum_buf,
        dsem_p,
        dsem_m,
        osem,
        ysend,
        yrecv,
        zsend,
        zrecv,
    ):
        my_x = lax.axis_index("x")
        my_y = lax.axis_index("y")
        my_z = lax.axis_index("z")
        ypeer = (my_x, 1 - my_y, my_z)
        znb = (my_x, my_y, 1 - my_z)

        row0 = my_z * ROWS
        my_col = my_y * HALF
        peer_col = (1 - my_y) * HALF

        dmas_p = []
        dmas_m = []
        for c in range(NCH):
            rows = pl.ds(row0 + c * CROWS, CROWS)
            dp = pltpu.make_async_copy(
                x_ref.at[0, rows, pl.ds(peer_col, HALF)], in_peer.at[c], dsem_p.at[c]
            )
            dp.start()
            dmas_p.append(dp)
            dm = pltpu.make_async_copy(
                x_ref.at[0, rows, pl.ds(my_col, HALF)], in_mine.at[c], dsem_m.at[c]
            )
            dm.start()
            dmas_m.append(dm)

        barrier = pltpu.get_barrier_semaphore()
        for nbr in (ypeer, znb):
            pl.semaphore_signal(
                barrier, inc=1, device_id=nbr, device_id_type=pl.DeviceIdType.MESH
            )
        pl.semaphore_wait(barrier, 2)

        rdmas_y = []
        for c in range(NCH):
            dmas_p[c].wait()
            send_y[c] = in_peer[c].astype(jnp.bfloat16)
            ry = pltpu.make_async_remote_copy(
                src_ref=send_y.at[c],
                dst_ref=recv_y.at[c],
                send_sem=ysend.at[c],
                recv_sem=yrecv.at[c],
                device_id=ypeer,
                device_id_type=pl.DeviceIdType.MESH,
            )
            ry.start()
            rdmas_y.append(ry)

        rdmas_z = []
        copies = []
        for c in range(NCH):
            dmas_m[c].wait()
            rdmas_y[c].wait()
            rows = pl.ds(row0 + c * CROWS, CROWS)
            sum_buf[c] = in_mine[c].astype(jnp.bfloat16) + recv_y[c]
            rz = pltpu.make_async_remote_copy(
                src_ref=sum_buf.at[c],
                dst_ref=out_ref.at[rows],
                send_sem=zsend.at[c],
                recv_sem=zrecv.at[c],
                device_id=znb,
                device_id_type=pl.DeviceIdType.MESH,
            )
            rz.start()
            rdmas_z.append(rz)
            cp = pltpu.make_async_copy(sum_buf.at[c], out_ref.at[rows], osem.at[c])
            cp.start()
            copies.append(cp)

        for c in range(NCH):
            copies[c].wait()
            rdmas_z[c].wait()

    return pl.pallas_call(
        body,
        out_shape=jax.ShapeDtypeStruct((M, HALF), jnp.bfloat16),
        in_specs=[pl.BlockSpec(memory_space=pl.ANY)],
        out_specs=pl.BlockSpec(memory_space=pl.ANY),
        scratch_shapes=[
            pltpu.VMEM((NCH, CROWS, HALF), jnp.float32),
            pltpu.VMEM((NCH, CROWS, HALF), jnp.float32),
            pltpu.VMEM((NCH, CROWS, HALF), jnp.bfloat16),
            pltpu.VMEM((NCH, CROWS, HALF), jnp.bfloat16),
            pltpu.VMEM((NCH, CROWS, HALF), jnp.bfloat16),
            pltpu.SemaphoreType.DMA((NCH,)),
            pltpu.SemaphoreType.DMA((NCH,)),
            pltpu.SemaphoreType.DMA((NCH,)),
            pltpu.SemaphoreType.DMA((NCH,)),
            pltpu.SemaphoreType.DMA((NCH,)),
            pltpu.SemaphoreType.DMA((NCH,)),
            pltpu.SemaphoreType.DMA((NCH,)),
        ],
        compiler_params=pltpu.CompilerParams(collective_id=0),
    )(x)


# device time: 16255 ns/iter; 1.1418x vs baseline; 1.0286x over previous
import jax
import jax.numpy as jnp
from jax import lax
from jax.experimental import pallas as pl
from jax.experimental.pallas import tpu as pltpu

M = 1024
HALF = 512
ROWS = 512
NCH = 8
CROWS = ROWS // NCH


def kernel(x):
    def body(
        x_ref,
        out_ref,
        in_all,
        send_y,
        recv_y,
        sum_buf,
        dsem,
        osem,
        ysend,
        yrecv,
        zsend,
        zrecv,
    ):
        my_x = lax.axis_index("x")
        my_y = lax.axis_index("y")
        my_z = lax.axis_index("z")
        ypeer = (my_x, 1 - my_y, my_z)
        znb = (my_x, my_y, 1 - my_z)

        row0 = my_z * ROWS
        my_col = my_y * HALF
        peer_col = (1 - my_y) * HALF

        barrier = pltpu.get_barrier_semaphore()
        for nbr in (ypeer, znb):
            pl.semaphore_signal(
                barrier, inc=1, device_id=nbr, device_id_type=pl.DeviceIdType.MESH
            )

        dmas = []
        for c in range(NCH):
            rows = pl.ds(row0 + c * CROWS, CROWS)
            dm = pltpu.make_async_copy(x_ref.at[0, rows, :], in_all.at[c], dsem.at[c])
            dm.start()
            dmas.append(dm)

        pl.semaphore_wait(barrier, 2)

        rdmas_y = []
        for c in range(NCH):
            dmas[c].wait()
            send_y[c] = in_all[c, :, pl.ds(peer_col, HALF)].astype(jnp.bfloat16)
            ry = pltpu.make_async_remote_copy(
                src_ref=send_y.at[c],
                dst_ref=recv_y.at[c],
                send_sem=ysend.at[c],
                recv_sem=yrecv.at[c],
                device_id=ypeer,
                device_id_type=pl.DeviceIdType.MESH,
            )
            ry.start()
            rdmas_y.append(ry)

        rdmas_z = []
        copies = []
        for c in range(NCH):
            rdmas_y[c].wait()
            rows = pl.ds(row0 + c * CROWS, CROWS)
            sum_buf[c] = (
                in_all[c, :, pl.ds(my_col, HALF)].astype(jnp.bfloat16) + recv_y[c]
            )
            rz = pltpu.make_async_remote_copy(
                src_ref=sum_buf.at[c],
                dst_ref=out_ref.at[rows],
                send_sem=zsend.at[c],
                recv_sem=zrecv.at[c],
                device_id=znb,
                device_id_type=pl.DeviceIdType.MESH,
            )
            rz.start()
            rdmas_z.append(rz)
            cp = pltpu.make_async_copy(sum_buf.at[c], out_ref.at[rows], osem.at[c])
            cp.start()
            copies.append(cp)

        for c in range(NCH):
            copies[c].wait()
            rdmas_z[c].wait()

    return pl.pallas_call(
        body,
        out_shape=jax.ShapeDtypeStruct((M, HALF), jnp.bfloat16),
        in_specs=[pl.BlockSpec(memory_space=pl.ANY)],
        out_specs=pl.BlockSpec(memory_space=pl.ANY),
        scratch_shapes=[
            pltpu.VMEM((NCH, CROWS, 2 * HALF), jnp.float32),
            pltpu.VMEM((NCH, CROWS, HALF), jnp.bfloat16),
            pltpu.VMEM((NCH, CROWS, HALF), jnp.bfloat16),
            pltpu.VMEM((NCH, CROWS, HALF), jnp.bfloat16),
            pltpu.SemaphoreType.DMA((NCH,)),
            pltpu.SemaphoreType.DMA((NCH,)),
            pltpu.SemaphoreType.DMA((NCH,)),
            pltpu.SemaphoreType.DMA((NCH,)),
            pltpu.SemaphoreType.DMA((NCH,)),
            pltpu.SemaphoreType.DMA((NCH,)),
        ],
        compiler_params=pltpu.CompilerParams(collective_id=0),
    )(x)


# device time: 15866 ns/iter; 1.1698x vs baseline; 1.0245x over previous
import jax
import jax.numpy as jnp
from jax import lax
from jax.experimental import pallas as pl
from jax.experimental.pallas import tpu as pltpu

M = 1024
HALF = 512
ROWS = 512
CROWS = 64
NSELF = ROWS // CROWS
EXTRA = 2
NPULL = NSELF + EXTRA
NFWD = NSELF - EXTRA


def kernel(x):
    def body(
        x_ref,
        out_ref,
        in_all,
        send_y,
        recv_y,
        sum_buf,
        dsem,
        osem,
        ysend,
        yrecv,
        zsend,
        zrecv,
    ):
        my_x = lax.axis_index("x")
        my_y = lax.axis_index("y")
        my_z = lax.axis_index("z")
        ypeer = (my_x, 1 - my_y, my_z)
        znb = (my_x, my_y, 1 - my_z)

        row0 = my_z * ROWS
        other0 = (1 - my_z) * ROWS
        my_col = my_y * HALF
        peer_col = (1 - my_y) * HALF

        def chunk_row(c):
            if c < NSELF:
                return row0 + c * CROWS
            return other0 + (NSELF - EXTRA + (c - NSELF)) * CROWS

        barrier = pltpu.get_barrier_semaphore()
        for nbr in (ypeer, znb):
            pl.semaphore_signal(
                barrier, inc=1, device_id=nbr, device_id_type=pl.DeviceIdType.MESH
            )

        dmas = []
        for c in range(NPULL):
            rows = pl.ds(chunk_row(c), CROWS)
            dm = pltpu.make_async_copy(x_ref.at[0, rows, :], in_all.at[c], dsem.at[c])
            dm.start()
            dmas.append(dm)

        pl.semaphore_wait(barrier, 2)

        rdmas_y = []
        for c in range(NPULL):
            dmas[c].wait()
            send_y[c] = in_all[c, :, pl.ds(peer_col, HALF)].astype(jnp.bfloat16)
            ry = pltpu.make_async_remote_copy(
                src_ref=send_y.at[c],
                dst_ref=recv_y.at[c],
                send_sem=ysend.at[c],
                recv_sem=yrecv.at[c],
                device_id=ypeer,
                device_id_type=pl.DeviceIdType.MESH,
            )
            ry.start()
            rdmas_y.append(ry)

        rdmas_z = []
        copies = []
        for c in range(NPULL):
            rdmas_y[c].wait()
            rows = pl.ds(chunk_row(c), CROWS)
            sum_buf[c] = (
                in_all[c, :, pl.ds(my_col, HALF)].astype(jnp.bfloat16) + recv_y[c]
            )
            if c < NFWD:
                rz = pltpu.make_async_remote_copy(
                    src_ref=sum_buf.at[c],
                    dst_ref=out_ref.at[rows],
                    send_sem=zsend.at[c],
                    recv_sem=zrecv.at[c],
                    device_id=znb,
                    device_id_type=pl.DeviceIdType.MESH,
                )
                rz.start()
                rdmas_z.append(rz)
            cp = pltpu.make_async_copy(sum_buf.at[c], out_ref.at[rows], osem.at[c])
            cp.start()
            copies.append(cp)

        for cp in copies:
            cp.wait()
        for rz in rdmas_z:
            rz.wait()

    return pl.pallas_call(
        body,
        out_shape=jax.ShapeDtypeStruct((M, HALF), jnp.bfloat16),
        in_specs=[pl.BlockSpec(memory_space=pl.ANY)],
        out_specs=pl.BlockSpec(memory_space=pl.ANY),
        scratch_shapes=[
            pltpu.VMEM((NPULL, CROWS, 2 * HALF), jnp.float32),
            pltpu.VMEM((NPULL, CROWS, HALF), jnp.bfloat16),
            pltpu.VMEM((NPULL, CROWS, HALF), jnp.bfloat16),
            pltpu.VMEM((NPULL, CROWS, HALF), jnp.bfloat16),
            pltpu.SemaphoreType.DMA((NPULL,)),
            pltpu.SemaphoreType.DMA((NPULL,)),
            pltpu.SemaphoreType.DMA((NPULL,)),
            pltpu.SemaphoreType.DMA((NPULL,)),
            pltpu.SemaphoreType.DMA((NFWD,)),
            pltpu.SemaphoreType.DMA((NFWD,)),
        ],
        compiler_params=pltpu.CompilerParams(collective_id=0),
    )(x)
